# Initial kernel scaffold; baseline (speedup 1.0000x reference)
import jax
import jax.numpy as jnp
from jax.experimental import pallas as pl


def kernel(ability, labels, wkr_idx, rel_idx, tsk_idx, w_relation, bias):
    raise NotImplementedError("write your pallas kernel here")


# speedup vs baseline: 7.2869x; 7.2869x over previous
import functools

import jax
import jax.numpy as jnp
from jax import lax
from jax.experimental import pallas as pl
from jax.experimental.pallas import tpu as pltpu
from jax.experimental.pallas import tpu_sc as plsc

_NC = 2
_NS = 16
_NW = _NC * _NS
_L = 16

_NUM_RELS = 10
_INV_DENOM = 1.0 / (_NUM_RELS - 1)


def _sigmoid_table(ability, w_relation, bias):
    n, d = ability.shape
    br = 4000
    assert n % br == 0

    def body(a_ref, w_ref, b_ref, o_ref):
        x = jnp.dot(a_ref[...], w_ref[...], preferred_element_type=jnp.float32)
        o_ref[...] = jax.nn.sigmoid(x + b_ref[0])

    return pl.pallas_call(
        body,
        grid=(n // br,),
        in_specs=[
            pl.BlockSpec((br, d), lambda i: (i, 0)),
            pl.BlockSpec((d, 1), lambda i: (0, 0)),
            pl.BlockSpec(memory_space=pltpu.SMEM),
        ],
        out_specs=pl.BlockSpec((br, 1), lambda i: (i, 0)),
        out_shape=jax.ShapeDtypeStruct((n, 1), jnp.float32),
    )(ability, w_relation, bias)


@functools.lru_cache(maxsize=None)
def _edge_kernel(e_pad, chunk, n_chunks):
    per_tile = chunk * n_chunks
    mesh = plsc.VectorSubcoreMesh(core_axis_name="c", subcore_axis_name="s")

    @functools.partial(
        pl.kernel,
        out_type=jax.ShapeDtypeStruct((e_pad,), jnp.float32),
        mesh=mesh,
        scratch_types=[
            pltpu.VMEM((chunk,), jnp.int32),
            pltpu.VMEM((chunk,), jnp.int32),
            pltpu.VMEM((chunk,), jnp.int32),
            pltpu.VMEM((chunk,), jnp.int32),
            pltpu.VMEM((chunk,), jnp.float32),
            pltpu.VMEM((chunk,), jnp.float32),
            pltpu.VMEM((chunk,), jnp.float32),
            pltpu.SemaphoreType.DMA,
            pltpu.SemaphoreType.DMA,
        ],
    )
    def body(p_hbm, lab_hbm, wkr_hbm, tsk_hbm, rel_hbm, out_hbm,
             wkr_v, tsk_v, rel_v, fidx_v, p_v, t_v, o_v, sem_p, sem_t):
        wid = lax.axis_index("s") * _NC + lax.axis_index("c")
        base = wid * per_tile
        for g in range(n_chunks):
            off = base + g * chunk
            pltpu.sync_copy(wkr_hbm.at[pl.ds(off, chunk)], wkr_v)
            pltpu.sync_copy(tsk_hbm.at[pl.ds(off, chunk)], tsk_v)
            pltpu.sync_copy(rel_hbm.at[pl.ds(off, chunk)], rel_v)

            def fidx_body(i, _):
                s = pl.ds(pl.multiple_of(i * _L, _L), _L)
                fidx_v[s] = tsk_v[s] * _NUM_RELS + rel_v[s]
                return 0

            lax.fori_loop(0, chunk // _L, fidx_body, 0)

            cp_p = pltpu.async_copy(p_hbm.at[wkr_v], p_v, sem_p)
            cp_t = pltpu.async_copy(lab_hbm.at[fidx_v], t_v, sem_t)
            cp_p.wait()
            cp_t.wait()

            def blend_body(i, _):
                s = pl.ds(pl.multiple_of(i * _L, _L), _L)
                p16 = p_v[s]
                t16 = t_v[s]
                q = (1.0 - p16) * _INV_DENOM
                o_v[s] = p16 * t16 + q * (1.0 - t16)
                return 0

            lax.fori_loop(0, chunk // _L, blend_body, 0)
            pltpu.sync_copy(o_v, out_hbm.at[pl.ds(off, chunk)])

    return body


def kernel(ability, labels, wkr_idx, rel_idx, tsk_idx, w_relation, bias):
    e = wkr_idx.shape[0]
    num_rels = labels.shape[2]
    assert num_rels == _NUM_RELS

    p_tab = _sigmoid_table(ability, w_relation, bias)[:, 0]
    lab_flat = labels.reshape(-1)

    chunk = 7840
    n_chunks = -(-e // (_NW * chunk))
    e_pad = _NW * chunk * n_chunks
    pad = e_pad - e

    wkr = jnp.pad(wkr_idx.astype(jnp.int32), (0, pad))
    tsk = jnp.pad(tsk_idx.astype(jnp.int32), (0, pad))
    rel = jnp.pad(rel_idx.astype(jnp.int32), (0, pad))

    out = _edge_kernel(e_pad, chunk, n_chunks)(p_tab, lab_flat, wkr, tsk, rel)
    return out[:e][:, None]


# speedup vs baseline: 8.1583x; 1.1196x over previous
import functools

import jax
import jax.numpy as jnp
from jax import lax
from jax.experimental import pallas as pl
from jax.experimental.pallas import tpu as pltpu
from jax.experimental.pallas import tpu_sc as plsc

_NC = 2
_NS = 16
_NW = _NC * _NS
_L = 16

_NUM_RELS = 10
_INV_DENOM = 1.0 / (_NUM_RELS - 1)


def _sigmoid_table(ability, w_relation, bias):
    n, d = ability.shape
    br = 4000
    assert n % br == 0

    def body(a_ref, w_ref, b_ref, o_ref):
        x = jnp.dot(a_ref[...], w_ref[...], preferred_element_type=jnp.float32)
        o_ref[...] = jax.nn.sigmoid(x + b_ref[0])

    return pl.pallas_call(
        body,
        grid=(n // br,),
        in_specs=[
            pl.BlockSpec((br, d), lambda i: (i, 0)),
            pl.BlockSpec((d, 1), lambda i: (0, 0)),
            pl.BlockSpec(memory_space=pltpu.SMEM),
        ],
        out_specs=pl.BlockSpec((br, 1), lambda i: (i, 0)),
        out_shape=jax.ShapeDtypeStruct((n, 1), jnp.float32),
    )(ability, w_relation, bias)


def _blend(p16, t16):
    q = (1.0 - p16) * _INV_DENOM
    return p16 * t16 + q * (1.0 - t16)


@functools.lru_cache(maxsize=None)
def _edge_kernel(e, chunk, n_per_tile, tail):
    mesh = plsc.VectorSubcoreMesh(core_axis_name="c", subcore_axis_name="s")
    n_vec = chunk // _L
    unroll = 8
    assert n_vec % unroll == 0

    vm_i = lambda: pltpu.VMEM((chunk,), jnp.int32)
    vm_f = lambda: pltpu.VMEM((chunk,), jnp.float32)
    scratch = [vm_i() for _ in range(6)] + [vm_f() for _ in range(6)] + \
              [pltpu.SemaphoreType.DMA for _ in range(8)]

    @functools.partial(
        pl.kernel,
        out_type=jax.ShapeDtypeStruct((e,), jnp.float32),
        mesh=mesh,
        scratch_types=scratch,
    )
    def body(p_hbm, lab_hbm, wkr_hbm, tsk_hbm, rel_hbm, out_hbm,
             wkr0, wkr1, tsk0, tsk1, fid0, fid1,
             p0, p1, t0, t1, o0, o1,
             si0, si1, sp0, sp1, st0, st1, so0, so1):
        wkr_b, tsk_b, fid_b = [wkr0, wkr1], [tsk0, tsk1], [fid0, fid1]
        p_b, t_b, o_b = [p0, p1], [t0, t1], [o0, o1]
        sem_i, sem_p, sem_t, sem_o = [si0, si1], [sp0, sp1], [st0, st1], [so0, so1]

        wid = lax.axis_index("s") * _NC + lax.axis_index("c")

        def load_idx(g, b):
            off = (g * _NW) * chunk + wid * chunk
            return (pltpu.async_copy(wkr_hbm.at[pl.ds(off, chunk)], wkr_b[b], sem_i[b]),
                    pltpu.async_copy(tsk_hbm.at[pl.ds(off, chunk)], tsk_b[b], sem_i[b]),
                    pltpu.async_copy(rel_hbm.at[pl.ds(off, chunk)], fid_b[b], sem_i[b]))

        def fidx_loop(b):
            def fbody(i, _):
                s = pl.ds(pl.multiple_of(i * _L, _L), _L)
                fid_b[b][s] = tsk_b[b][s] * _NUM_RELS + fid_b[b][s]
                return 0
            lax.fori_loop(0, n_vec, fbody, 0, unroll=unroll)

        def blend_loop(b):
            def bbody(i, _):
                s = pl.ds(pl.multiple_of(i * _L, _L), _L)
                o_b[b][s] = _blend(p_b[b][s], t_b[b][s])
                return 0
            lax.fori_loop(0, n_vec, bbody, 0, unroll=unroll)

        idx_cps = {0: load_idx(0, 0)}
        gat_cps = {}
        out_cps = {}
        for g in range(n_per_tile):
            b, nb = g % 2, (g + 1) % 2
            for cp in idx_cps.pop(g):
                cp.wait()
            fidx_loop(b)
            gat_cps[g] = (
                pltpu.async_copy(p_hbm.at[wkr_b[b]], p_b[b], sem_p[b]),
                pltpu.async_copy(lab_hbm.at[fid_b[b]], t_b[b], sem_t[b]),
            )
            if g >= 1:
                for cp in gat_cps.pop(g - 1):
                    cp.wait()
            if g + 1 < n_per_tile:
                idx_cps[g + 1] = load_idx(g + 1, nb)
            if g >= 1:
                if g - 1 in out_cps:
                    out_cps.pop(g - 1).wait()
                blend_loop(nb)
                off = ((g - 1) * _NW) * chunk + wid * chunk
                out_cps[g - 1] = pltpu.async_copy(
                    o_b[nb], out_hbm.at[pl.ds(off, chunk)], sem_o[nb])
        gl = n_per_tile - 1
        bl = gl % 2
        for cp in gat_cps.pop(gl):
            cp.wait()
        if gl - 1 in out_cps:
            out_cps.pop(gl - 1).wait()
        blend_loop(bl)
        off = (gl * _NW) * chunk + wid * chunk
        pltpu.sync_copy(o_b[bl], out_hbm.at[pl.ds(off, chunk)])

        if tail:
            t_off = n_per_tile * _NW * chunk

            @pl.when(wid == 0)
            def _():
                sl = pl.ds(0, tail)
                pltpu.sync_copy(wkr_hbm.at[pl.ds(t_off, tail)], wkr0.at[sl])
                pltpu.sync_copy(tsk_hbm.at[pl.ds(t_off, tail)], tsk0.at[sl])
                pltpu.sync_copy(rel_hbm.at[pl.ds(t_off, tail)], fid0.at[sl])

                def fbody(i, _):
                    s = pl.ds(pl.multiple_of(i * _L, _L), _L)
                    fid0[s] = tsk0[s] * _NUM_RELS + fid0[s]
                    return 0
                lax.fori_loop(0, tail // _L, fbody, 0, unroll=4)

                cp_p = pltpu.async_copy(p_hbm.at[wkr0.at[sl]], p0.at[sl], sp0)
                cp_t = pltpu.async_copy(lab_hbm.at[fid0.at[sl]], t0.at[sl], st0)
                cp_p.wait()
                cp_t.wait()

                def bbody(i, _):
                    s = pl.ds(pl.multiple_of(i * _L, _L), _L)
                    o0[s] = _blend(p0[s], t0[s])
                    return 0
                lax.fori_loop(0, tail // _L, bbody, 0, unroll=4)
                pltpu.sync_copy(o0.at[sl], out_hbm.at[pl.ds(t_off, tail)])

    return body


def kernel(ability, labels, wkr_idx, rel_idx, tsk_idx, w_relation, bias):
    e = wkr_idx.shape[0]
    assert labels.shape[2] == _NUM_RELS

    p_tab = _sigmoid_table(ability, w_relation, bias)[:, 0]
    lab_flat = labels.reshape(-1)

    chunk = 7808
    n_per_tile = e // (_NW * chunk)
    tail = e - _NW * chunk * n_per_tile
    assert n_per_tile >= 2 and tail < chunk and tail % _L == 0

    out = _edge_kernel(e, chunk, n_per_tile, tail)(
        p_tab, lab_flat,
        wkr_idx.astype(jnp.int32), tsk_idx.astype(jnp.int32),
        rel_idx.astype(jnp.int32))
    return out.reshape(e, 1)


# speedup vs baseline: 12.4048x; 1.5205x over previous
import functools

import jax
import jax.numpy as jnp
from jax import lax
from jax.experimental import pallas as pl
from jax.experimental.pallas import tpu as pltpu
from jax.experimental.pallas import tpu_sc as plsc

_NC = 2
_NS = 16
_NW = _NC * _NS
_L = 16

_NUM_RELS = 10
_INV_DENOM = 1.0 / (_NUM_RELS - 1)


def _sigmoid_table(ability, w_relation, bias):
    n, d = ability.shape
    at = ability.T
    br = 4096

    def body(a_ref, w_ref, b_ref, o_ref):
        x = jnp.sum(a_ref[...] * w_ref[...], axis=0) + b_ref[0]
        o_ref[...] = jax.nn.sigmoid(x)

    return pl.pallas_call(
        body,
        grid=(-(-n // br),),
        in_specs=[
            pl.BlockSpec((d, br), lambda i: (0, i)),
            pl.BlockSpec((d, 1), lambda i: (0, 0)),
            pl.BlockSpec(memory_space=pltpu.SMEM),
        ],
        out_specs=pl.BlockSpec((br,), lambda i: (i,)),
        out_shape=jax.ShapeDtypeStruct((n,), jnp.float32),
    )(at, w_relation, bias)


def _blend(p16, t16):
    q = (1.0 - p16) * _INV_DENOM
    return p16 * t16 + q * (1.0 - t16)


@functools.lru_cache(maxsize=None)
def _edge_kernel(e, n_tsk, chunk, n_per_tile, tail):
    mesh = plsc.VectorSubcoreMesh(core_axis_name="c", subcore_axis_name="s")
    n_vec = chunk // _L
    unroll = 8
    assert n_vec % unroll == 0

    vm_i = lambda: pltpu.VMEM((chunk,), jnp.int32)
    vm_f = lambda: pltpu.VMEM((chunk,), jnp.float32)
    scratch = [vm_i() for _ in range(6)] + [vm_f() for _ in range(6)] + \
              [pltpu.SemaphoreType.DMA for _ in range(8)]

    @functools.partial(
        pl.kernel,
        out_type=jax.ShapeDtypeStruct((e,), jnp.float32),
        mesh=mesh,
        scratch_types=scratch,
    )
    def body(p_hbm, lab_hbm, wkr_hbm, tsk_hbm, rel_hbm, out_hbm,
             wkr0, wkr1, tsk0, tsk1, fid0, fid1,
             p0, p1, t0, t1, o0, o1,
             si0, si1, sp0, sp1, st0, st1, so0, so1):
        wkr_b, tsk_b, fid_b = [wkr0, wkr1], [tsk0, tsk1], [fid0, fid1]
        p_b, t_b, o_b = [p0, p1], [t0, t1], [o0, o1]
        sem_i, sem_p, sem_t, sem_o = [si0, si1], [sp0, sp1], [st0, st1], [so0, so1]

        wid = lax.axis_index("s") * _NC + lax.axis_index("c")

        def load_idx(g, b):
            off = (g * _NW) * chunk + wid * chunk
            return (pltpu.async_copy(wkr_hbm.at[pl.ds(off, chunk)], wkr_b[b], sem_i[b]),
                    pltpu.async_copy(tsk_hbm.at[pl.ds(off, chunk)], tsk_b[b], sem_i[b]),
                    pltpu.async_copy(rel_hbm.at[pl.ds(off, chunk)], fid_b[b], sem_i[b]))

        def fidx_loop(b):
            def fbody(i, _):
                s = pl.ds(pl.multiple_of(i * _L, _L), _L)
                fid_b[b][s] = fid_b[b][s] * n_tsk + tsk_b[b][s]
                return 0
            lax.fori_loop(0, n_vec, fbody, 0, unroll=unroll)

        def blend_loop(b):
            def bbody(i, _):
                s = pl.ds(pl.multiple_of(i * _L, _L), _L)
                o_b[b][s] = _blend(p_b[b][s], t_b[b][s])
                return 0
            lax.fori_loop(0, n_vec, bbody, 0, unroll=unroll)

        idx_cps = {0: load_idx(0, 0)}
        gat_cps = {}
        out_cps = {}
        for g in range(n_per_tile):
            b, nb = g % 2, (g + 1) % 2
            for cp in idx_cps.pop(g):
                cp.wait()
            fidx_loop(b)
            gat_cps[g] = (
                pltpu.async_copy(p_hbm.at[wkr_b[b]], p_b[b], sem_p[b]),
                pltpu.async_copy(lab_hbm.at[fid_b[b]], t_b[b], sem_t[b]),
            )
            if g >= 1:
                for cp in gat_cps.pop(g - 1):
                    cp.wait()
            if g + 1 < n_per_tile:
                idx_cps[g + 1] = load_idx(g + 1, nb)
            if g >= 1:
                if g - 1 in out_cps:
                    out_cps.pop(g - 1).wait()
                blend_loop(nb)
                off = ((g - 1) * _NW) * chunk + wid * chunk
                out_cps[g - 1] = pltpu.async_copy(
                    o_b[nb], out_hbm.at[pl.ds(off, chunk)], sem_o[nb])
        gl = n_per_tile - 1
        bl = gl % 2
        for cp in gat_cps.pop(gl):
            cp.wait()
        if gl - 1 in out_cps:
            out_cps.pop(gl - 1).wait()
        blend_loop(bl)
        off = (gl * _NW) * chunk + wid * chunk
        pltpu.sync_copy(o_b[bl], out_hbm.at[pl.ds(off, chunk)])

        if tail:
            t_off = n_per_tile * _NW * chunk

            @pl.when(wid == 0)
            def _():
                sl = pl.ds(0, tail)
                pltpu.sync_copy(wkr_hbm.at[pl.ds(t_off, tail)], wkr0.at[sl])
                pltpu.sync_copy(tsk_hbm.at[pl.ds(t_off, tail)], tsk0.at[sl])
                pltpu.sync_copy(rel_hbm.at[pl.ds(t_off, tail)], fid0.at[sl])

                def fbody(i, _):
                    s = pl.ds(pl.multiple_of(i * _L, _L), _L)
                    fid0[s] = fid0[s] * n_tsk + tsk0[s]
                    return 0
                lax.fori_loop(0, tail // _L, fbody, 0, unroll=4)

                cp_p = pltpu.async_copy(p_hbm.at[wkr0.at[sl]], p0.at[sl], sp0)
                cp_t = pltpu.async_copy(lab_hbm.at[fid0.at[sl]], t0.at[sl], st0)
                cp_p.wait()
                cp_t.wait()

                def bbody(i, _):
                    s = pl.ds(pl.multiple_of(i * _L, _L), _L)
                    o0[s] = _blend(p0[s], t0[s])
                    return 0
                lax.fori_loop(0, tail // _L, bbody, 0, unroll=4)
                pltpu.sync_copy(o0.at[sl], out_hbm.at[pl.ds(t_off, tail)])

    return body


def kernel(ability, labels, wkr_idx, rel_idx, tsk_idx, w_relation, bias):
    e = wkr_idx.shape[0]
    assert labels.shape[2] == _NUM_RELS

    n_tsk = labels.shape[0]
    p_tab = _sigmoid_table(ability, w_relation, bias)
    lab_flat = labels.transpose(2, 1, 0).reshape(-1)

    chunk = 7808
    n_per_tile = e // (_NW * chunk)
    tail = e - _NW * chunk * n_per_tile
    assert n_per_tile >= 2 and tail < chunk and tail % _L == 0

    out = _edge_kernel(e, n_tsk, chunk, n_per_tile, tail)(
        p_tab, lab_flat,
        wkr_idx.astype(jnp.int32), tsk_idx.astype(jnp.int32),
        rel_idx.astype(jnp.int32))
    return out.reshape(e, 1)


# speedup vs baseline: 16.3767x; 1.3202x over previous
import functools

import jax
import jax.numpy as jnp
from jax import lax
from jax.experimental import pallas as pl
from jax.experimental.pallas import tpu as pltpu
from jax.experimental.pallas import tpu_sc as plsc

_NC = 2
_NS = 16
_NW = _NC * _NS
_L = 16

_NUM_RELS = 10
_INV_DENOM = 1.0 / (_NUM_RELS - 1)


def _sigmoid_table(ability, w_relation, bias):
    n, d = ability.shape
    at = ability.T
    br = 4096

    def body(a_ref, w_ref, b_ref, o_ref):
        x = jnp.sum(a_ref[...] * w_ref[...], axis=0) + b_ref[0]
        o_ref[...] = jax.nn.sigmoid(x)

    return pl.pallas_call(
        body,
        grid=(-(-n // br),),
        in_specs=[
            pl.BlockSpec((d, br), lambda i: (0, i)),
            pl.BlockSpec((d, 1), lambda i: (0, 0)),
            pl.BlockSpec(memory_space=pltpu.SMEM),
        ],
        out_specs=pl.BlockSpec((br,), lambda i: (i,)),
        out_shape=jax.ShapeDtypeStruct((n,), jnp.float32),
    )(at, w_relation, bias)


def _blend(p16, t16):
    q = (1.0 - p16) * _INV_DENOM
    return p16 * t16 + q * (1.0 - t16)


@functools.lru_cache(maxsize=None)
def _edge_kernel(e, n_wkr, n_lab, chunk, n_per_tile, tail):
    mesh = plsc.VectorSubcoreMesh(core_axis_name="c", subcore_axis_name="s")
    n_vec = chunk // _L
    unroll = next(u for u in (8, 4, 2, 1) if n_vec % u == 0)
    n_tab = n_wkr + n_lab

    per_p = (n_wkr // _NS) & ~7
    per_lab = (n_lab // _NS) & ~7
    p_rem = n_wkr - _NS * per_p
    lab_rem = n_lab - _NS * per_lab
    assert p_rem % 8 == 0 and lab_rem % 8 == 0
    assert p_rem <= chunk and lab_rem <= chunk

    vm_i = lambda: pltpu.VMEM((chunk,), jnp.int32)
    vm_f = lambda: pltpu.VMEM((chunk,), jnp.float32)
    scratch = [vm_i() for _ in range(6)] + [vm_f() for _ in range(4)] + \
              [pltpu.VMEM_SHARED((n_tab,), jnp.float32)] + \
              [pltpu.SemaphoreType.DMA for _ in range(8)]

    @functools.partial(
        pl.kernel,
        out_type=jax.ShapeDtypeStruct((e,), jnp.float32),
        mesh=mesh,
        scratch_types=scratch,
    )
    def body(p_hbm, lab_hbm, wkr_hbm, tsk_hbm, rel_hbm, out_hbm,
             wkr0, wkr1, tsk0, tsk1, fid0, fid1,
             p0, p1, t0, t1, tab,
             si0, si1, sp0, sp1, st0, st1, so0, so1):
        wkr_b, tsk_b, fid_b = [wkr0, wkr1], [tsk0, tsk1], [fid0, fid1]
        p_b, t_b = [p0, p1], [t0, t1]
        sem_i, sem_p, sem_t, sem_o = [si0, si1], [sp0, sp1], [st0, st1], [so0, so1]

        sid = lax.axis_index("s")
        wid = sid * _NC + lax.axis_index("c")

        bounce = [p0, p1]

        def seg_src(kind, so, w):
            return (p_hbm if kind == 0 else lab_hbm).at[pl.ds(so, w)]

        segs = []
        for kind, per, dst_base in ((0, per_p, 0), (1, per_lab, n_wkr)):
            pos = 0
            while pos < per:
                w = min(chunk, per - pos)
                segs.append((kind, sid * per + pos, dst_base + sid * per + pos, w))
                pos += w
        n_main = len(segs)
        if p_rem:
            segs.append((0, _NS * per_p, _NS * per_p, p_rem))
        if lab_rem:
            segs.append((1, _NS * per_lab, n_wkr + _NS * per_lab, lab_rem))

        cps = {}
        for k, (kind, so, do, w) in enumerate(segs):
            rem = k >= n_main

            def issue(kind=kind, so=so, w=w, k=k):
                return pltpu.async_copy(
                    seg_src(kind, so, w), bounce[k % 2].at[pl.ds(0, w)],
                    sem_p[k % 2])

            if rem:
                @pl.when(sid == 0)
                def _(issue=issue, k=k):
                    cps[k] = issue()
            else:
                cps[k] = issue()
            if k - 1 in cps:
                def hop2(k=k, prev=segs[k - 1]):
                    cp = cps.pop(k - 1)
                    cp.wait()
                    _, _, do1, w1 = prev
                    pltpu.sync_copy(bounce[(k - 1) % 2].at[pl.ds(0, w1)],
                                    tab.at[pl.ds(do1, w1)])
                if k - 1 >= n_main:
                    @pl.when(sid == 0)
                    def _(hop2=hop2):
                        hop2()
                else:
                    hop2()
        kl = len(segs) - 1

        def hop2_last():
            cp = cps.pop(kl)
            cp.wait()
            _, _, do1, w1 = segs[kl]
            pltpu.sync_copy(bounce[kl % 2].at[pl.ds(0, w1)],
                            tab.at[pl.ds(do1, w1)])

        if kl >= n_main:
            @pl.when(sid == 0)
            def _():
                hop2_last()
        else:
            hop2_last()

        def load_idx(g, b):
            off = (g * _NW) * chunk + wid * chunk
            return (pltpu.async_copy(wkr_hbm.at[pl.ds(off, chunk)], wkr_b[b], sem_i[b]),
                    pltpu.async_copy(tsk_hbm.at[pl.ds(off, chunk)], tsk_b[b], sem_i[b]),
                    pltpu.async_copy(rel_hbm.at[pl.ds(off, chunk)], fid_b[b], sem_i[b]))

        def fidx_loop(b):
            def fbody(i, _):
                s = pl.ds(pl.multiple_of(i * _L, _L), _L)
                fid_b[b][s] = fid_b[b][s] * (n_lab // _NUM_RELS) + tsk_b[b][s] + n_wkr
                return 0
            lax.fori_loop(0, n_vec, fbody, 0, unroll=unroll)

        def blend_loop(b):
            def bbody(i, _):
                s = pl.ds(pl.multiple_of(i * _L, _L), _L)
                p_b[b][s] = _blend(p_b[b][s], t_b[b][s])
                return 0
            lax.fori_loop(0, n_vec, bbody, 0, unroll=unroll)

        idx_cps = {0: load_idx(0, 0)}
        gat_cps = {}
        out_cps = {}
        for g in range(n_per_tile):
            b, nb = g % 2, (g + 1) % 2
            for cp in idx_cps.pop(g):
                cp.wait()
            fidx_loop(b)
            if g == 0:
                plsc.subcore_barrier()
            if g - 2 in out_cps:
                out_cps.pop(g - 2).wait()
            gat_cps[g] = (
                pltpu.async_copy(tab.at[wkr_b[b]], p_b[b], sem_p[b]),
                pltpu.async_copy(tab.at[fid_b[b]], t_b[b], sem_t[b]),
            )
            if g >= 1:
                for cp in gat_cps.pop(g - 1):
                    cp.wait()
            if g + 1 < n_per_tile:
                idx_cps[g + 1] = load_idx(g + 1, nb)
            if g >= 1:
                blend_loop(nb)
                off = ((g - 1) * _NW) * chunk + wid * chunk
                out_cps[g - 1] = pltpu.async_copy(
                    p_b[nb], out_hbm.at[pl.ds(off, chunk)], sem_o[nb])
        gl = n_per_tile - 1
        bl = gl % 2
        for cp in gat_cps.pop(gl):
            cp.wait()
        if gl - 1 in out_cps:
            out_cps.pop(gl - 1).wait()
        blend_loop(bl)
        off = (gl * _NW) * chunk + wid * chunk
        pltpu.sync_copy(p_b[bl], out_hbm.at[pl.ds(off, chunk)])

        if tail:
            t_off = n_per_tile * _NW * chunk

            @pl.when(wid == 0)
            def _():
                sl = pl.ds(0, tail)
                pltpu.sync_copy(wkr_hbm.at[pl.ds(t_off, tail)], wkr0.at[sl])
                pltpu.sync_copy(tsk_hbm.at[pl.ds(t_off, tail)], tsk0.at[sl])
                pltpu.sync_copy(rel_hbm.at[pl.ds(t_off, tail)], fid0.at[sl])

                def fbody(i, _):
                    s = pl.ds(pl.multiple_of(i * _L, _L), _L)
                    fid0[s] = fid0[s] * (n_lab // _NUM_RELS) + tsk0[s] + n_wkr
                    return 0
                lax.fori_loop(0, tail // _L, fbody, 0, unroll=4)

                cp_p = pltpu.async_copy(tab.at[wkr0.at[sl]], p0.at[sl], sp0)
                cp_t = pltpu.async_copy(tab.at[fid0.at[sl]], t0.at[sl], st0)
                cp_p.wait()
                cp_t.wait()

                def bbody(i, _):
                    s = pl.ds(pl.multiple_of(i * _L, _L), _L)
                    p0[s] = _blend(p0[s], t0[s])
                    return 0
                lax.fori_loop(0, tail // _L, bbody, 0, unroll=4)
                pltpu.sync_copy(p0.at[sl], out_hbm.at[pl.ds(t_off, tail)])

    return body


def kernel(ability, labels, wkr_idx, rel_idx, tsk_idx, w_relation, bias):
    e = wkr_idx.shape[0]
    assert labels.shape[2] == _NUM_RELS

    n_tsk = labels.shape[0]
    n_wkr = ability.shape[0]
    p_tab = _sigmoid_table(ability, w_relation, bias)
    lab_flat = labels.transpose(2, 1, 0).reshape(-1)

    chunk = 5184
    n_per_tile = e // (_NW * chunk)
    tail = e - _NW * chunk * n_per_tile
    assert n_per_tile >= 2 and tail < chunk and tail % _L == 0

    out = _edge_kernel(e, n_wkr, n_tsk * _NUM_RELS, chunk, n_per_tile, tail)(
        p_tab, lab_flat,
        wkr_idx.astype(jnp.int32), tsk_idx.astype(jnp.int32),
        rel_idx.astype(jnp.int32))
    return out.reshape(e, 1)


# speedup vs baseline: 20.3437x; 1.2422x over previous
import functools

import jax
import jax.numpy as jnp
from jax import lax
from jax.experimental import pallas as pl
from jax.experimental.pallas import tpu as pltpu
from jax.experimental.pallas import tpu_sc as plsc

_NC = 2
_NS = 16
_NW = _NC * _NS
_L = 16

_NUM_RELS = 10
_INV_DENOM = 1.0 / (_NUM_RELS - 1)


def _sigmoid_table(ability, w_relation, bias):
    n, d = ability.shape
    at = ability.T
    br = 4096

    def body(a_ref, w_ref, b_ref, o_ref):
        x = jnp.sum(a_ref[...] * w_ref[...], axis=0) + b_ref[0]
        o_ref[...] = jax.nn.sigmoid(x)

    return pl.pallas_call(
        body,
        grid=(-(-n // br),),
        in_specs=[
            pl.BlockSpec((d, br), lambda i: (0, i)),
            pl.BlockSpec((d, 1), lambda i: (0, 0)),
            pl.BlockSpec(memory_space=pltpu.SMEM),
        ],
        out_specs=pl.BlockSpec((br,), lambda i: (i,)),
        out_shape=jax.ShapeDtypeStruct((n,), jnp.float32),
    )(at, w_relation, bias)


def _blend(p16, t16):
    q = (1.0 - p16) * _INV_DENOM
    return p16 * t16 + q * (1.0 - t16)


@functools.lru_cache(maxsize=None)
def _edge_kernel(e, n_wkr, n_rel, n_tsk, chunk, n_per_tile, tail):
    n_lab = n_rel * n_tsk
    mesh = plsc.VectorSubcoreMesh(core_axis_name="c", subcore_axis_name="s")
    n_vec = chunk // _L
    unroll = next(u for u in (8, 4, 2, 1) if n_vec % u == 0)
    n_tab = n_wkr + n_lab

    per_p = (n_wkr // _NS) & ~7
    per_row = (n_tsk // _NS) & ~7
    p_rem = n_wkr - _NS * per_p
    row_rem = n_tsk - _NS * per_row
    assert p_rem % 8 == 0 and row_rem % 8 == 0
    assert p_rem <= chunk and row_rem <= chunk

    vm_i = lambda: pltpu.VMEM((chunk,), jnp.int32)
    vm_f = lambda: pltpu.VMEM((chunk,), jnp.float32)
    scratch = [vm_i() for _ in range(6)] + [vm_f() for _ in range(4)] + \
              [pltpu.VMEM_SHARED((n_tab,), jnp.float32)] + \
              [pltpu.SemaphoreType.DMA for _ in range(8)]

    @functools.partial(
        pl.kernel,
        out_type=jax.ShapeDtypeStruct((1, e), jnp.float32),
        mesh=mesh,
        scratch_types=scratch,
        compiler_params=pltpu.CompilerParams(use_tc_tiling_on_sc=False),
    )
    def body(p_hbm, lab_hbm, wkr_hbm, tsk_hbm, rel_hbm, out_hbm,
             wkr0, wkr1, tsk0, tsk1, fid0, fid1,
             p0, p1, t0, t1, tab,
             si0, si1, sp0, sp1, st0, st1, so0, so1):
        wkr_b, tsk_b, fid_b = [wkr0, wkr1], [tsk0, tsk1], [fid0, fid1]
        p_b, t_b = [p0, p1], [t0, t1]
        sem_i, sem_p, sem_t, sem_o = [si0, si1], [sp0, sp1], [st0, st1], [so0, so1]

        sid = lax.axis_index("s")
        wid = sid * _NC + lax.axis_index("c")

        bounce = [p0, p1]

        def seg_src(kind, so, w):
            if kind < 0:
                return p_hbm.at[pl.ds(so, w)]
            return lab_hbm.at[kind, pl.ds(so, w)]

        segs = []
        pos = 0
        while pos < per_p:
            w = min(chunk, per_p - pos)
            segs.append((-1, sid * per_p + pos, sid * per_p + pos, w))
            pos += w
        for r in range(n_rel):
            base = n_wkr + r * n_tsk
            pos = 0
            while pos < per_row:
                w = min(chunk, per_row - pos)
                segs.append((r, sid * per_row + pos,
                             base + sid * per_row + pos, w))
                pos += w
        n_main = len(segs)
        if p_rem:
            segs.append((-1, _NS * per_p, _NS * per_p, p_rem))
        if row_rem:
            for r in range(n_rel):
                segs.append((r, _NS * per_row,
                             n_wkr + r * n_tsk + _NS * per_row, row_rem))

        cps = {}
        for k, (kind, so, do, w) in enumerate(segs):
            rem = k >= n_main

            def issue(kind=kind, so=so, w=w, k=k):
                return pltpu.async_copy(
                    seg_src(kind, so, w), bounce[k % 2].at[pl.ds(0, w)],
                    sem_p[k % 2])

            if rem:
                @pl.when(sid == 0)
                def _(issue=issue, k=k):
                    cps[k] = issue()
            else:
                cps[k] = issue()
            if k - 1 in cps:
                def hop2(k=k, prev=segs[k - 1]):
                    cp = cps.pop(k - 1)
                    cp.wait()
                    _, _, do1, w1 = prev
                    pltpu.sync_copy(bounce[(k - 1) % 2].at[pl.ds(0, w1)],
                                    tab.at[pl.ds(do1, w1)])
                if k - 1 >= n_main:
                    @pl.when(sid == 0)
                    def _(hop2=hop2):
                        hop2()
                else:
                    hop2()
        kl = len(segs) - 1

        def hop2_last():
            cp = cps.pop(kl)
            cp.wait()
            _, _, do1, w1 = segs[kl]
            pltpu.sync_copy(bounce[kl % 2].at[pl.ds(0, w1)],
                            tab.at[pl.ds(do1, w1)])

        if kl >= n_main:
            @pl.when(sid == 0)
            def _():
                hop2_last()
        else:
            hop2_last()

        def load_idx(g, b):
            off = (g * _NW) * chunk + wid * chunk
            return (pltpu.async_copy(wkr_hbm.at[pl.ds(off, chunk)], wkr_b[b], sem_i[b]),
                    pltpu.async_copy(tsk_hbm.at[pl.ds(off, chunk)], tsk_b[b], sem_i[b]),
                    pltpu.async_copy(rel_hbm.at[pl.ds(off, chunk)], fid_b[b], sem_i[b]))

        def fidx_loop(b):
            def fbody(i, _):
                s = pl.ds(pl.multiple_of(i * _L, _L), _L)
                fid_b[b][s] = fid_b[b][s] * (n_lab // _NUM_RELS) + tsk_b[b][s] + n_wkr
                return 0
            lax.fori_loop(0, n_vec, fbody, 0, unroll=unroll)

        def blend_loop(b):
            def bbody(i, _):
                s = pl.ds(pl.multiple_of(i * _L, _L), _L)
                p_b[b][s] = _blend(p_b[b][s], t_b[b][s])
                return 0
            lax.fori_loop(0, n_vec, bbody, 0, unroll=unroll)

        idx_cps = {0: load_idx(0, 0)}
        gat_cps = {}
        out_cps = {}
        for g in range(n_per_tile):
            b, nb = g % 2, (g + 1) % 2
            for cp in idx_cps.pop(g):
                cp.wait()
            fidx_loop(b)
            if g == 0:
                plsc.subcore_barrier()
            if g - 2 in out_cps:
                out_cps.pop(g - 2).wait()
            gat_cps[g] = (
                pltpu.async_copy(tab.at[wkr_b[b]], p_b[b], sem_p[b]),
                pltpu.async_copy(tab.at[fid_b[b]], t_b[b], sem_t[b]),
            )
            if g >= 1:
                for cp in gat_cps.pop(g - 1):
                    cp.wait()
            if g + 1 < n_per_tile:
                idx_cps[g + 1] = load_idx(g + 1, nb)
            if g >= 1:
                blend_loop(nb)
                off = ((g - 1) * _NW) * chunk + wid * chunk
                out_cps[g - 1] = pltpu.async_copy(
                    p_b[nb], out_hbm.at[0, pl.ds(off, chunk)], sem_o[nb])
        gl = n_per_tile - 1
        bl = gl % 2
        for cp in gat_cps.pop(gl):
            cp.wait()
        if gl - 1 in out_cps:
            out_cps.pop(gl - 1).wait()
        blend_loop(bl)
        off = (gl * _NW) * chunk + wid * chunk
        pltpu.sync_copy(p_b[bl], out_hbm.at[0, pl.ds(off, chunk)])

        if tail:
            t_off = n_per_tile * _NW * chunk

            @pl.when(wid == 0)
            def _():
                sl = pl.ds(0, tail)
                pltpu.sync_copy(wkr_hbm.at[pl.ds(t_off, tail)], wkr0.at[sl])
                pltpu.sync_copy(tsk_hbm.at[pl.ds(t_off, tail)], tsk0.at[sl])
                pltpu.sync_copy(rel_hbm.at[pl.ds(t_off, tail)], fid0.at[sl])

                def fbody(i, _):
                    s = pl.ds(pl.multiple_of(i * _L, _L), _L)
                    fid0[s] = fid0[s] * (n_lab // _NUM_RELS) + tsk0[s] + n_wkr
                    return 0
                lax.fori_loop(0, tail // _L, fbody, 0, unroll=4)

                cp_p = pltpu.async_copy(tab.at[wkr0.at[sl]], p0.at[sl], sp0)
                cp_t = pltpu.async_copy(tab.at[fid0.at[sl]], t0.at[sl], st0)
                cp_p.wait()
                cp_t.wait()

                def bbody(i, _):
                    s = pl.ds(pl.multiple_of(i * _L, _L), _L)
                    p0[s] = _blend(p0[s], t0[s])
                    return 0
                lax.fori_loop(0, tail // _L, bbody, 0, unroll=4)
                pltpu.sync_copy(p0.at[sl], out_hbm.at[0, pl.ds(t_off, tail)])

    return body


def kernel(ability, labels, wkr_idx, rel_idx, tsk_idx, w_relation, bias):
    e = wkr_idx.shape[0]
    assert labels.shape[2] == _NUM_RELS

    n_tsk = labels.shape[0]
    n_wkr = ability.shape[0]
    p_tab = _sigmoid_table(ability, w_relation, bias)
    lab2 = labels.transpose(2, 1, 0).reshape(_NUM_RELS, n_tsk)

    chunk = 5184
    n_per_tile = e // (_NW * chunk)
    tail = e - _NW * chunk * n_per_tile
    assert n_per_tile >= 2 and tail < chunk and tail % _L == 0

    out = _edge_kernel(e, n_wkr, _NUM_RELS, n_tsk, chunk, n_per_tile, tail)(
        p_tab, lab2,
        wkr_idx.astype(jnp.int32), tsk_idx.astype(jnp.int32),
        rel_idx.astype(jnp.int32))
    return out.T


# speedup vs baseline: 21.3804x; 1.0510x over previous
import functools

import jax
import jax.numpy as jnp
from jax import lax
from jax.experimental import pallas as pl
from jax.experimental.pallas import tpu as pltpu
from jax.experimental.pallas import tpu_sc as plsc

_NC = 2
_NS = 16
_NW = _NC * _NS
_L = 16

_NUM_RELS = 10
_INV_DENOM = 1.0 / (_NUM_RELS - 1)


def _sigmoid_table(ability, w_relation, bias):
    n, d = ability.shape
    at = ability.T
    br = 8192

    def body(a_ref, w_ref, b_ref, o_ref):
        x = jnp.sum(a_ref[...] * w_ref[...], axis=0) + b_ref[0]
        o_ref[...] = jax.nn.sigmoid(x)

    return pl.pallas_call(
        body,
        grid=(-(-n // br),),
        in_specs=[
            pl.BlockSpec((d, br), lambda i: (0, i)),
            pl.BlockSpec((d, 1), lambda i: (0, 0)),
            pl.BlockSpec(memory_space=pltpu.SMEM),
        ],
        out_specs=pl.BlockSpec((br,), lambda i: (i,)),
        out_shape=jax.ShapeDtypeStruct((n,), jnp.float32),
    )(at, w_relation, bias)


def _blend(p16, t16):
    q = (1.0 - p16) * _INV_DENOM
    return p16 * t16 + q * (1.0 - t16)


@functools.lru_cache(maxsize=None)
def _edge_kernel(e, n_wkr, n_rel, n_tsk, chunk, n_per_tile, tail):
    n_lab = n_rel * n_tsk
    mesh = plsc.VectorSubcoreMesh(core_axis_name="c", subcore_axis_name="s")
    n_vec = chunk // _L
    unroll = next(u for u in (8, 4, 2, 1) if n_vec % u == 0)
    n_tab = n_wkr + n_lab

    per_p = (n_wkr // _NS) & ~7
    per_row = (n_tsk // _NS) & ~7
    p_rem = n_wkr - _NS * per_p
    row_rem = n_tsk - _NS * per_row
    assert p_rem % 8 == 0 and row_rem % 8 == 0
    assert p_rem <= chunk and row_rem <= chunk

    vm_i = lambda: pltpu.VMEM((chunk,), jnp.int32)
    vm_f = lambda: pltpu.VMEM((chunk,), jnp.float32)
    scratch = [vm_i() for _ in range(6)] + [vm_f() for _ in range(4)] + \
              [pltpu.VMEM_SHARED((n_tab,), jnp.float32)] + \
              [pltpu.SemaphoreType.DMA for _ in range(8)]

    @functools.partial(
        pl.kernel,
        out_type=jax.ShapeDtypeStruct((1, e), jnp.float32),
        mesh=mesh,
        scratch_types=scratch,
        compiler_params=pltpu.CompilerParams(use_tc_tiling_on_sc=False),
    )
    def body(p_hbm, lab_hbm, wkr_hbm, tsk_hbm, rel_hbm, out_hbm,
             wkr0, wkr1, tsk0, tsk1, fid0, fid1,
             p0, p1, t0, t1, tab,
             si0, si1, sp0, sp1, st0, st1, so0, so1):
        wkr_b, tsk_b, fid_b = [wkr0, wkr1], [tsk0, tsk1], [fid0, fid1]
        p_b, t_b = [p0, p1], [t0, t1]
        sem_i, sem_p, sem_t, sem_o = [si0, si1], [sp0, sp1], [st0, st1], [so0, so1]

        sid = lax.axis_index("s")
        wid = sid * _NC + lax.axis_index("c")

        bounce = [p0, p1]

        def seg_src(kind, so, w):
            if kind < 0:
                return p_hbm.at[pl.ds(so, w)]
            return lab_hbm.at[kind, pl.ds(so, w)]

        segs = []
        pos = 0
        while pos < per_p:
            w = min(chunk, per_p - pos)
            segs.append((-1, sid * per_p + pos, sid * per_p + pos, w))
            pos += w
        for r in range(n_rel):
            base = n_wkr + r * n_tsk
            pos = 0
            while pos < per_row:
                w = min(chunk, per_row - pos)
                segs.append((r, sid * per_row + pos,
                             base + sid * per_row + pos, w))
                pos += w
        n_main = len(segs)
        if p_rem:
            segs.append((-1, _NS * per_p, _NS * per_p, p_rem))
        if row_rem:
            for r in range(n_rel):
                segs.append((r, _NS * per_row,
                             n_wkr + r * n_tsk + _NS * per_row, row_rem))

        cps = {}
        for k, (kind, so, do, w) in enumerate(segs):
            rem = k >= n_main

            def issue(kind=kind, so=so, w=w, k=k):
                return pltpu.async_copy(
                    seg_src(kind, so, w), bounce[k % 2].at[pl.ds(0, w)],
                    sem_p[k % 2])

            if rem:
                @pl.when(sid == 0)
                def _(issue=issue, k=k):
                    cps[k] = issue()
            else:
                cps[k] = issue()
            if k - 1 in cps:
                def hop2(k=k, prev=segs[k - 1]):
                    cp = cps.pop(k - 1)
                    cp.wait()
                    _, _, do1, w1 = prev
                    pltpu.sync_copy(bounce[(k - 1) % 2].at[pl.ds(0, w1)],
                                    tab.at[pl.ds(do1, w1)])
                if k - 1 >= n_main:
                    @pl.when(sid == 0)
                    def _(hop2=hop2):
                        hop2()
                else:
                    hop2()
        kl = len(segs) - 1

        def hop2_last():
            cp = cps.pop(kl)
            cp.wait()
            _, _, do1, w1 = segs[kl]
            pltpu.sync_copy(bounce[kl % 2].at[pl.ds(0, w1)],
                            tab.at[pl.ds(do1, w1)])

        if kl >= n_main:
            @pl.when(sid == 0)
            def _():
                hop2_last()
        else:
            hop2_last()

        def load_idx(g, b):
            off = (g * _NW) * chunk + wid * chunk
            return (pltpu.async_copy(wkr_hbm.at[pl.ds(off, chunk)], wkr_b[b], sem_i[b]),
                    pltpu.async_copy(tsk_hbm.at[pl.ds(off, chunk)], tsk_b[b], sem_i[b]),
                    pltpu.async_copy(rel_hbm.at[pl.ds(off, chunk)], fid_b[b], sem_i[b]))

        def fidx_loop(b):
            def fbody(i, _):
                s = pl.ds(pl.multiple_of(i * _L, _L), _L)
                fid_b[b][s] = fid_b[b][s] * (n_lab // _NUM_RELS) + tsk_b[b][s] + n_wkr
                return 0
            lax.fori_loop(0, n_vec, fbody, 0, unroll=unroll)

        def blend_loop(b):
            def bbody(i, _):
                s = pl.ds(pl.multiple_of(i * _L, _L), _L)
                p_b[b][s] = _blend(p_b[b][s], t_b[b][s])
                return 0
            lax.fori_loop(0, n_vec, bbody, 0, unroll=unroll)

        idx_cps = {0: load_idx(0, 0)}
        gat_cps = {}
        out_cps = {}
        for g in range(n_per_tile):
            b, nb = g % 2, (g + 1) % 2
            for cp in idx_cps.pop(g):
                cp.wait()
            fidx_loop(b)
            if g == 0:
                plsc.subcore_barrier()
            if g - 2 in out_cps:
                out_cps.pop(g - 2).wait()
            gat_cps[g] = (
                pltpu.async_copy(tab.at[wkr_b[b]], p_b[b], sem_p[b]),
                pltpu.async_copy(tab.at[fid_b[b]], t_b[b], sem_t[b]),
            )
            if g >= 1:
                for cp in gat_cps.pop(g - 1):
                    cp.wait()
            if g + 1 < n_per_tile:
                idx_cps[g + 1] = load_idx(g + 1, nb)
            if g >= 1:
                blend_loop(nb)
                off = ((g - 1) * _NW) * chunk + wid * chunk
                out_cps[g - 1] = pltpu.async_copy(
                    p_b[nb], out_hbm.at[0, pl.ds(off, chunk)], sem_o[nb])
        gl = n_per_tile - 1
        bl = gl % 2
        for cp in gat_cps.pop(gl):
            cp.wait()
        if gl - 1 in out_cps:
            out_cps.pop(gl - 1).wait()
        blend_loop(bl)
        off = (gl * _NW) * chunk + wid * chunk
        pltpu.sync_copy(p_b[bl], out_hbm.at[0, pl.ds(off, chunk)])

        if tail:
            t_off = n_per_tile * _NW * chunk

            @pl.when(wid == 0)
            def _():
                sl = pl.ds(0, tail)
                pltpu.sync_copy(wkr_hbm.at[pl.ds(t_off, tail)], wkr0.at[sl])
                pltpu.sync_copy(tsk_hbm.at[pl.ds(t_off, tail)], tsk0.at[sl])
                pltpu.sync_copy(rel_hbm.at[pl.ds(t_off, tail)], fid0.at[sl])

                def fbody(i, _):
                    s = pl.ds(pl.multiple_of(i * _L, _L), _L)
                    fid0[s] = fid0[s] * (n_lab // _NUM_RELS) + tsk0[s] + n_wkr
                    return 0
                lax.fori_loop(0, tail // _L, fbody, 0, unroll=4)

                cp_p = pltpu.async_copy(tab.at[wkr0.at[sl]], p0.at[sl], sp0)
                cp_t = pltpu.async_copy(tab.at[fid0.at[sl]], t0.at[sl], st0)
                cp_p.wait()
                cp_t.wait()

                def bbody(i, _):
                    s = pl.ds(pl.multiple_of(i * _L, _L), _L)
                    p0[s] = _blend(p0[s], t0[s])
                    return 0
                lax.fori_loop(0, tail // _L, bbody, 0, unroll=4)
                pltpu.sync_copy(p0.at[sl], out_hbm.at[0, pl.ds(t_off, tail)])

    return body


def kernel(ability, labels, wkr_idx, rel_idx, tsk_idx, w_relation, bias):
    e = wkr_idx.shape[0]
    assert labels.shape[2] == _NUM_RELS

    n_tsk = labels.shape[0]
    n_wkr = ability.shape[0]
    p_tab = _sigmoid_table(ability, w_relation, bias)
    lab2 = labels.transpose(2, 1, 0).reshape(_NUM_RELS, n_tsk)

    chunk = 5184
    n_per_tile = e // (_NW * chunk)
    tail = e - _NW * chunk * n_per_tile
    assert n_per_tile >= 2 and tail < chunk and tail % _L == 0

    out = _edge_kernel(e, n_wkr, _NUM_RELS, n_tsk, chunk, n_per_tile, tail)(
        p_tab, lab2,
        wkr_idx.astype(jnp.int32), tsk_idx.astype(jnp.int32),
        rel_idx.astype(jnp.int32))
    return out.T


# speedup vs baseline: 24.2551x; 1.1345x over previous
import functools

import jax
import jax.numpy as jnp
from jax import lax
from jax.experimental import pallas as pl
from jax.experimental.pallas import tpu as pltpu
from jax.experimental.pallas import tpu_sc as plsc

_NC = 2
_NS = 16
_NW = _NC * _NS
_L = 16

_NUM_RELS = 10
_INV_DENOM = 1.0 / (_NUM_RELS - 1)


def _sigmoid_table(ability, w_relation, bias):
    n, d = ability.shape
    at = ability.T
    br = 8192

    def body(a_ref, w_ref, b_ref, o_ref):
        x = jnp.sum(a_ref[...] * w_ref[...], axis=0) + b_ref[0]
        o_ref[...] = jax.nn.sigmoid(x)

    return pl.pallas_call(
        body,
        grid=(-(-n // br),),
        in_specs=[
            pl.BlockSpec((d, br), lambda i: (0, i)),
            pl.BlockSpec((d, 1), lambda i: (0, 0)),
            pl.BlockSpec(memory_space=pltpu.SMEM),
        ],
        out_specs=pl.BlockSpec((br,), lambda i: (i,)),
        out_shape=jax.ShapeDtypeStruct((n,), jnp.float32),
    )(at, w_relation, bias)


def _blend(p16, t16):
    q = (1.0 - p16) * _INV_DENOM
    return p16 * t16 + q * (1.0 - t16)


@functools.lru_cache(maxsize=None)
def _edge_kernel(e, n_wkr, n_rel, n_tsk, chunk, n_per_tile, tail):
    n_lab = n_rel * n_tsk
    mesh = plsc.VectorSubcoreMesh(core_axis_name="c", subcore_axis_name="s")
    n_vec = chunk // _L
    unroll = next(u for u in (8, 4, 2, 1) if n_vec % u == 0)
    n_tab = n_wkr + n_lab

    per_p = (n_wkr // _NS) & ~7
    per_row = (n_tsk // _NS) & ~7
    p_rem = n_wkr - _NS * per_p
    row_rem = n_tsk - _NS * per_row
    assert p_rem % 8 == 0 and row_rem % 8 == 0
    assert p_rem <= chunk and row_rem <= chunk

    vm_i = lambda: pltpu.VMEM((chunk,), jnp.int32)
    vm_f = lambda: pltpu.VMEM((chunk,), jnp.float32)
    scratch = [vm_i() for _ in range(6)] + [vm_f() for _ in range(4)] + \
              [pltpu.VMEM_SHARED((n_tab,), jnp.float32)] + \
              [pltpu.SemaphoreType.DMA for _ in range(8)]

    @functools.partial(
        pl.kernel,
        out_type=jax.ShapeDtypeStruct((1, e), jnp.float32),
        mesh=mesh,
        scratch_types=scratch,
        compiler_params=pltpu.CompilerParams(use_tc_tiling_on_sc=False),
    )
    def body(p_hbm, lab_hbm, wkr_hbm, tsk_hbm, rel_hbm, out_hbm,
             wkr0, wkr1, tsk0, tsk1, fid0, fid1,
             p0, p1, t0, t1, tab,
             si0, si1, sp0, sp1, st0, st1, so0, so1):
        wkr_b, tsk_b, fid_b = [wkr0, wkr1], [tsk0, tsk1], [fid0, fid1]
        p_b, t_b = [p0, p1], [t0, t1]
        sem_i, sem_p, sem_t, sem_o = [si0, si1], [sp0, sp1], [st0, st1], [so0, so1]

        sid = lax.axis_index("s")
        wid = sid * _NC + lax.axis_index("c")
        stage_sems = [sp0, sp1, st0, st1]

        bounce = [p0, p1, t0, t1]

        def seg_src(kind, so, w):
            if kind < 0:
                return p_hbm.at[pl.ds(so, w)]
            return lab_hbm.at[kind, pl.ds(so, w)]

        segs = []
        pos = 0
        while pos < per_p:
            w = min(chunk, per_p - pos)
            segs.append((-1, sid * per_p + pos, sid * per_p + pos, w))
            pos += w
        for r in range(n_rel):
            base = n_wkr + r * n_tsk
            pos = 0
            while pos < per_row:
                w = min(chunk, per_row - pos)
                segs.append((r, sid * per_row + pos,
                             base + sid * per_row + pos, w))
                pos += w
        n_main = len(segs)
        if p_rem:
            segs.append((-1, _NS * per_p, _NS * per_p, p_rem))
        if row_rem:
            for r in range(n_rel):
                segs.append((r, _NS * per_row,
                             n_wkr + r * n_tsk + _NS * per_row, row_rem))

        cps = {}
        for k, (kind, so, do, w) in enumerate(segs):
            rem = k >= n_main

            def issue(kind=kind, so=so, w=w, k=k):
                return pltpu.async_copy(
                    seg_src(kind, so, w), bounce[k % 4].at[pl.ds(0, w)],
                    stage_sems[k % 4])

            if rem:
                @pl.when(sid == 0)
                def _(issue=issue, k=k):
                    cps[k] = issue()
            else:
                cps[k] = issue()
            if k - 1 in cps:
                def hop2(k=k, prev=segs[k - 1]):
                    cp = cps.pop(k - 1)
                    cp.wait()
                    _, _, do1, w1 = prev
                    pltpu.sync_copy(bounce[(k - 1) % 4].at[pl.ds(0, w1)],
                                    tab.at[pl.ds(do1, w1)])
                if k - 1 >= n_main:
                    @pl.when(sid == 0)
                    def _(hop2=hop2):
                        hop2()
                else:
                    hop2()
        kl = len(segs) - 1

        def hop2_last():
            cp = cps.pop(kl)
            cp.wait()
            _, _, do1, w1 = segs[kl]
            pltpu.sync_copy(bounce[kl % 4].at[pl.ds(0, w1)],
                            tab.at[pl.ds(do1, w1)])

        if kl >= n_main:
            @pl.when(sid == 0)
            def _():
                hop2_last()
        else:
            hop2_last()

        def load_idx(g, b):
            off = (g * _NW) * chunk + wid * chunk
            return (pltpu.async_copy(wkr_hbm.at[pl.ds(off, chunk)], wkr_b[b], sem_i[b]),
                    pltpu.async_copy(tsk_hbm.at[pl.ds(off, chunk)], tsk_b[b], sem_i[b]),
                    pltpu.async_copy(rel_hbm.at[pl.ds(off, chunk)], fid_b[b], sem_i[b]))

        def fidx_loop(b):
            @plsc.parallel_loop(0, chunk, step=_L, unroll=unroll)
            def _(i):
                s = pl.ds(pl.multiple_of(i, _L), _L)
                fid_b[b][s] = fid_b[b][s] * (n_lab // _NUM_RELS) + tsk_b[b][s] + n_wkr

        def blend_loop(b):
            @plsc.parallel_loop(0, chunk, step=_L, unroll=unroll)
            def _(i):
                s = pl.ds(pl.multiple_of(i, _L), _L)
                p_b[b][s] = _blend(p_b[b][s], t_b[b][s])

        idx_cps = {0: load_idx(0, 0)}
        gat_cps = {}
        out_cps = {}
        for g in range(n_per_tile):
            b, nb = g % 2, (g + 1) % 2
            for cp in idx_cps.pop(g):
                cp.wait()
            fidx_loop(b)
            if g == 0:
                plsc.subcore_barrier()
            if g - 2 in out_cps:
                out_cps.pop(g - 2).wait()
            gat_cps[g] = (
                pltpu.async_copy(tab.at[wkr_b[b]], p_b[b], sem_p[b]),
                pltpu.async_copy(tab.at[fid_b[b]], t_b[b], sem_t[b]),
            )
            if g >= 1:
                for cp in gat_cps.pop(g - 1):
                    cp.wait()
            if g + 1 < n_per_tile:
                idx_cps[g + 1] = load_idx(g + 1, nb)
            if g >= 1:
                blend_loop(nb)
                off = ((g - 1) * _NW) * chunk + wid * chunk
                out_cps[g - 1] = pltpu.async_copy(
                    p_b[nb], out_hbm.at[0, pl.ds(off, chunk)], sem_o[nb])
        gl = n_per_tile - 1
        bl = gl % 2
        for cp in gat_cps.pop(gl):
            cp.wait()
        if gl - 1 in out_cps:
            out_cps.pop(gl - 1).wait()
        blend_loop(bl)
        off = (gl * _NW) * chunk + wid * chunk
        pltpu.sync_copy(p_b[bl], out_hbm.at[0, pl.ds(off, chunk)])

        if tail:
            t_off = n_per_tile * _NW * chunk
            t_per = -(-(tail // _L) // _NW) * _L
            n_full = tail // t_per
            t_rest = tail - n_full * t_per
            assert t_per % 8 == 0 and t_rest % _L == 0
            assert n_full <= _NW and (t_rest == 0 or n_full < _NW)

            def do_tail(my_off, width):
                sl = pl.ds(0, width)
                pltpu.sync_copy(wkr_hbm.at[pl.ds(my_off, width)], wkr0.at[sl])
                pltpu.sync_copy(tsk_hbm.at[pl.ds(my_off, width)], tsk0.at[sl])
                pltpu.sync_copy(rel_hbm.at[pl.ds(my_off, width)], fid0.at[sl])

                @plsc.parallel_loop(0, width, step=_L, unroll=1)
                def _(i):
                    s = pl.ds(pl.multiple_of(i, _L), _L)
                    fid0[s] = fid0[s] * (n_lab // _NUM_RELS) + tsk0[s] + n_wkr

                cp_p = pltpu.async_copy(tab.at[wkr0.at[sl]], p0.at[sl], sp0)
                cp_t = pltpu.async_copy(tab.at[fid0.at[sl]], t0.at[sl], st0)
                cp_p.wait()
                cp_t.wait()

                @plsc.parallel_loop(0, width, step=_L, unroll=1)
                def _(i):
                    s = pl.ds(pl.multiple_of(i, _L), _L)
                    p0[s] = _blend(p0[s], t0[s])
                pltpu.sync_copy(p0.at[sl], out_hbm.at[0, pl.ds(my_off, width)])

            @pl.when(wid < n_full)
            def _():
                do_tail(t_off + wid * t_per, t_per)
            if t_rest:
                @pl.when(wid == n_full)
                def _():
                    do_tail(t_off + n_full * t_per, t_rest)

    return body


def kernel(ability, labels, wkr_idx, rel_idx, tsk_idx, w_relation, bias):
    e = wkr_idx.shape[0]
    assert labels.shape[2] == _NUM_RELS

    n_tsk = labels.shape[0]
    n_wkr = ability.shape[0]
    p_tab = _sigmoid_table(ability, w_relation, bias)
    lab2 = labels.transpose(2, 1, 0).reshape(_NUM_RELS, n_tsk)

    chunk = 5184
    n_per_tile = e // (_NW * chunk)
    tail = e - _NW * chunk * n_per_tile
    assert n_per_tile >= 2 and tail < chunk and tail % _L == 0

    out = _edge_kernel(e, n_wkr, _NUM_RELS, n_tsk, chunk, n_per_tile, tail)(
        p_tab, lab2,
        wkr_idx.astype(jnp.int32), tsk_idx.astype(jnp.int32),
        rel_idx.astype(jnp.int32))
    return out.T
